# initial kernel scaffold (unmeasured)
import jax
import jax.numpy as jnp
from jax import lax
from jax.experimental import pallas as pl
from jax.experimental.pallas import tpu as pltpu


def kernel(
    x,
):
    def body(*refs):
        pass

    out_shape = jax.ShapeDtypeStruct(..., jnp.float32)
    return pl.pallas_call(body, out_shape=out_shape)(...)


# baseline (device time: 41678 ns/iter reference)
import jax
import jax.numpy as jnp
from jax import lax
from jax.experimental import pallas as pl
from jax.experimental.pallas import tpu as pltpu

N_Z = 4


def kernel(x):
    m, n = x.shape

    def body(x_ref, out_ref, comm_ref, send_sems, recv_sems):
        my_x = lax.axis_index("x")
        my_y = lax.axis_index("y")
        my_z = lax.axis_index("z")

        barrier_sem = pltpu.get_barrier_semaphore()
        for d in range(1, N_Z):
            peer = (my_z + d) % N_Z
            pl.semaphore_signal(
                barrier_sem, inc=1,
                device_id=(my_x, my_y, peer),
                device_id_type=pl.DeviceIdType.MESH,
            )
        pl.semaphore_wait(barrier_sem, N_Z - 1)

        rdmas = []
        for d in range(1, N_Z):
            peer = (my_z + d) % N_Z
            rdma = pltpu.make_async_remote_copy(
                src_ref=x_ref,
                dst_ref=comm_ref.at[d - 1],
                send_sem=send_sems.at[d - 1],
                recv_sem=recv_sems.at[d - 1],
                device_id=(my_x, my_y, peer),
                device_id_type=pl.DeviceIdType.MESH,
            )
            rdma.start()
            rdmas.append(rdma)
        for rdma in rdmas:
            rdma.wait()

        out_ref[...] = (
            x_ref[...] + comm_ref[0] + comm_ref[1] + comm_ref[2]
        )

    return pl.pallas_call(
        body,
        out_shape=jax.ShapeDtypeStruct((m, n), x.dtype),
        in_specs=[pl.BlockSpec(memory_space=pltpu.VMEM)],
        out_specs=pl.BlockSpec(memory_space=pltpu.VMEM),
        scratch_shapes=[
            pltpu.VMEM((N_Z - 1, m, n), x.dtype),
            pltpu.SemaphoreType.DMA((N_Z - 1,)),
            pltpu.SemaphoreType.DMA((N_Z - 1,)),
        ],
        compiler_params=pltpu.CompilerParams(collective_id=0),
    )(x)


# device time: 28144 ns/iter; 1.4809x vs baseline; 1.4809x over previous
import jax
import jax.numpy as jnp
from jax import lax
from jax.experimental import pallas as pl
from jax.experimental.pallas import tpu as pltpu

N_Z = 4


def kernel(x):
    m, n = x.shape
    mq = m // N_Z

    def body(x_ref, out_ref, rs_buf, red_buf, ag_buf,
             rs_send, rs_recv, ag_send, ag_recv):
        my_x = lax.axis_index("x")
        my_y = lax.axis_index("y")
        my_z = lax.axis_index("z")

        barrier_sem = pltpu.get_barrier_semaphore()
        for d in range(1, N_Z):
            peer = (my_z + d) % N_Z
            pl.semaphore_signal(
                barrier_sem, inc=1,
                device_id=(my_x, my_y, peer),
                device_id_type=pl.DeviceIdType.MESH,
            )
        pl.semaphore_wait(barrier_sem, N_Z - 1)

        rs = []
        for d in range(1, N_Z):
            peer = (my_z + d) % N_Z
            rdma = pltpu.make_async_remote_copy(
                src_ref=x_ref.at[pl.ds(peer * mq, mq)],
                dst_ref=rs_buf.at[d - 1],
                send_sem=rs_send.at[d - 1],
                recv_sem=rs_recv.at[d - 1],
                device_id=(my_x, my_y, peer),
                device_id_type=pl.DeviceIdType.MESH,
            )
            rdma.start()
            rs.append(rdma)
        for rdma in rs:
            rdma.wait()

        red_buf[...] = (
            x_ref[pl.ds(my_z * mq, mq), :]
            + rs_buf[0] + rs_buf[1] + rs_buf[2]
        )
        out_ref[pl.ds(my_z * mq, mq), :] = red_buf[...]

        ag = []
        for d in range(1, N_Z):
            peer = (my_z + d) % N_Z
            rdma = pltpu.make_async_remote_copy(
                src_ref=red_buf,
                dst_ref=ag_buf.at[d - 1],
                send_sem=ag_send.at[d - 1],
                recv_sem=ag_recv.at[d - 1],
                device_id=(my_x, my_y, peer),
                device_id_type=pl.DeviceIdType.MESH,
            )
            rdma.start()
            ag.append(rdma)
        for rdma in ag:
            rdma.wait()

        for s in range(N_Z - 1):
            origin = (my_z - s - 1) % N_Z
            out_ref[pl.ds(origin * mq, mq), :] = ag_buf[s]

    return pl.pallas_call(
        body,
        out_shape=jax.ShapeDtypeStruct((m, n), x.dtype),
        in_specs=[pl.BlockSpec(memory_space=pltpu.VMEM)],
        out_specs=pl.BlockSpec(memory_space=pltpu.VMEM),
        scratch_shapes=[
            pltpu.VMEM((N_Z - 1, mq, n), x.dtype),
            pltpu.VMEM((mq, n), x.dtype),
            pltpu.VMEM((N_Z - 1, mq, n), x.dtype),
            pltpu.SemaphoreType.DMA((N_Z - 1,)),
            pltpu.SemaphoreType.DMA((N_Z - 1,)),
            pltpu.SemaphoreType.DMA((N_Z - 1,)),
            pltpu.SemaphoreType.DMA((N_Z - 1,)),
        ],
        compiler_params=pltpu.CompilerParams(collective_id=0),
    )(x)


# device time: 26039 ns/iter; 1.6006x vs baseline; 1.0808x over previous
import jax
import jax.numpy as jnp
from jax import lax
from jax.experimental import pallas as pl
from jax.experimental.pallas import tpu as pltpu

N_X, N_Y, N_Z = 2, 2, 4

OFFS = [
    (dx, dy, dz)
    for dx in range(N_X)
    for dy in range(N_Y)
    for dz in range(N_Z)
    if (dx, dy, dz) != (0, 0, 0)
]


def kernel(x):
    m, n = x.shape
    tm = m // (N_X * N_Y)
    tn = n // N_Z

    def body(x_ref, out_ref, rs_buf, red_buf, ag_buf,
             rs_send, rs_recv, ag_send, ag_recv):
        my_x = lax.axis_index("x")
        my_y = lax.axis_index("y")
        my_z = lax.axis_index("z")
        r = 2 * my_x + my_y

        barrier_sem = pltpu.get_barrier_semaphore()
        for dx, dy, dz in OFFS:
            peer = ((my_x + dx) % N_X, (my_y + dy) % N_Y, (my_z + dz) % N_Z)
            pl.semaphore_signal(
                barrier_sem, inc=1,
                device_id=peer, device_id_type=pl.DeviceIdType.MESH,
            )
        pl.semaphore_wait(barrier_sem, len(OFFS))

        rs = []
        for d in range(1, N_Z):
            pz = (my_z + d) % N_Z
            rdma = pltpu.make_async_remote_copy(
                src_ref=x_ref.at[pl.ds(r * tm, tm), pl.ds(pz * tn, tn)],
                dst_ref=rs_buf.at[d - 1],
                send_sem=rs_send.at[d - 1],
                recv_sem=rs_recv.at[d - 1],
                device_id=(my_x, my_y, pz),
                device_id_type=pl.DeviceIdType.MESH,
            )
            rdma.start()
            rs.append(rdma)
        for rdma in rs:
            rdma.wait()

        red_buf[...] = (
            x_ref[pl.ds(r * tm, tm), pl.ds(my_z * tn, tn)]
            + rs_buf[0] + rs_buf[1] + rs_buf[2]
        )
        out_ref[pl.ds(r * tm, tm), pl.ds(my_z * tn, tn)] = red_buf[...]

        ag = []
        for s, (dx, dy, dz) in enumerate(OFFS):
            tgt = (
                (my_x + dx) % N_X,
                (my_y + dy) % N_Y,
                (my_z + N_Z - dz) % N_Z,
            )
            rdma = pltpu.make_async_remote_copy(
                src_ref=red_buf,
                dst_ref=ag_buf.at[s],
                send_sem=ag_send.at[s],
                recv_sem=ag_recv.at[s],
                device_id=tgt,
                device_id_type=pl.DeviceIdType.MESH,
            )
            rdma.start()
            ag.append(rdma)
        for rdma in ag:
            rdma.wait()

        for s, (dx, dy, dz) in enumerate(OFFS):
            ox = (my_x + dx) % N_X
            oy = (my_y + dy) % N_Y
            oz = (my_z + dz) % N_Z
            orow = 2 * ox + oy
            out_ref[pl.ds(orow * tm, tm), pl.ds(oz * tn, tn)] = ag_buf[s]

    return pl.pallas_call(
        body,
        out_shape=jax.ShapeDtypeStruct((m, n), x.dtype),
        in_specs=[pl.BlockSpec(memory_space=pltpu.VMEM)],
        out_specs=pl.BlockSpec(memory_space=pltpu.VMEM),
        scratch_shapes=[
            pltpu.VMEM((N_Z - 1, tm, tn), x.dtype),
            pltpu.VMEM((tm, tn), x.dtype),
            pltpu.VMEM((len(OFFS), tm, tn), x.dtype),
            pltpu.SemaphoreType.DMA((N_Z - 1,)),
            pltpu.SemaphoreType.DMA((N_Z - 1,)),
            pltpu.SemaphoreType.DMA((len(OFFS),)),
            pltpu.SemaphoreType.DMA((len(OFFS),)),
        ],
        compiler_params=pltpu.CompilerParams(collective_id=0),
    )(x)


# device time: 25960 ns/iter; 1.6055x vs baseline; 1.0030x over previous
import jax
import jax.numpy as jnp
from jax import lax
from jax.experimental import pallas as pl
from jax.experimental.pallas import tpu as pltpu

N_X, N_Y, N_Z = 2, 2, 4

OFFS = [
    (dx, dy, dz)
    for dx in range(N_X)
    for dy in range(N_Y)
    for dz in range(N_Z)
    if (dx, dy, dz) != (0, 0, 0)
]


def kernel(x):
    m, n = x.shape
    tm = m // (N_X * N_Y)
    tn = n // N_Z

    def body(x_ref, out_ref, rs_buf, red_buf, ag_buf,
             rs_send, rs_recv, ag_send, ag_recv):
        my_x = lax.axis_index("x")
        my_y = lax.axis_index("y")
        my_z = lax.axis_index("z")
        r = 2 * my_x + my_y

        with jax.named_scope("barrier"):
            barrier_sem = pltpu.get_barrier_semaphore()
            for dx, dy, dz in OFFS:
                peer = ((my_x + dx) % N_X, (my_y + dy) % N_Y, (my_z + dz) % N_Z)
                pl.semaphore_signal(
                    barrier_sem, inc=1,
                    device_id=peer, device_id_type=pl.DeviceIdType.MESH,
                )
            pl.semaphore_wait(barrier_sem, len(OFFS))

        with jax.named_scope("rs_start"):
            rs = []
            for d in range(1, N_Z):
                pz = (my_z + d) % N_Z
                rdma = pltpu.make_async_remote_copy(
                    src_ref=x_ref.at[pl.ds(r * tm, tm), pl.ds(pz * tn, tn)],
                    dst_ref=rs_buf.at[d - 1],
                    send_sem=rs_send.at[d - 1],
                    recv_sem=rs_recv.at[d - 1],
                    device_id=(my_x, my_y, pz),
                    device_id_type=pl.DeviceIdType.MESH,
                )
                rdma.start()
                rs.append(rdma)
        with jax.named_scope("rs_wait"):
            for rdma in rs:
                rdma.wait_recv()

        with jax.named_scope("reduce"):
            red_buf[...] = (
                x_ref[pl.ds(r * tm, tm), pl.ds(my_z * tn, tn)]
                + rs_buf[0] + rs_buf[1] + rs_buf[2]
            )
            out_ref[pl.ds(r * tm, tm), pl.ds(my_z * tn, tn)] = red_buf[...]

        with jax.named_scope("ag_start"):
            ag = []
            for s, (dx, dy, dz) in enumerate(OFFS):
                tgt = (
                    (my_x + dx) % N_X,
                    (my_y + dy) % N_Y,
                    (my_z + N_Z - dz) % N_Z,
                )
                rdma = pltpu.make_async_remote_copy(
                    src_ref=red_buf,
                    dst_ref=ag_buf.at[s],
                    send_sem=ag_send.at[s],
                    recv_sem=ag_recv.at[s],
                    device_id=tgt,
                    device_id_type=pl.DeviceIdType.MESH,
                )
                rdma.start()
                ag.append(rdma)
        with jax.named_scope("ag_wait_store"):
            for s, (dx, dy, dz) in enumerate(OFFS):
                ag[s].wait_recv()
                ox = (my_x + dx) % N_X
                oy = (my_y + dy) % N_Y
                oz = (my_z + dz) % N_Z
                orow = 2 * ox + oy
                out_ref[pl.ds(orow * tm, tm), pl.ds(oz * tn, tn)] = ag_buf[s]

        with jax.named_scope("send_cleanup"):
            for rdma in rs:
                rdma.wait_send()
            for rdma in ag:
                rdma.wait_send()

    return pl.pallas_call(
        body,
        out_shape=jax.ShapeDtypeStruct((m, n), x.dtype),
        in_specs=[pl.BlockSpec(memory_space=pltpu.VMEM)],
        out_specs=pl.BlockSpec(memory_space=pltpu.VMEM),
        scratch_shapes=[
            pltpu.VMEM((N_Z - 1, tm, tn), x.dtype),
            pltpu.VMEM((tm, tn), x.dtype),
            pltpu.VMEM((len(OFFS), tm, tn), x.dtype),
            pltpu.SemaphoreType.DMA((N_Z - 1,)),
            pltpu.SemaphoreType.DMA((N_Z - 1,)),
            pltpu.SemaphoreType.DMA((len(OFFS),)),
            pltpu.SemaphoreType.DMA((len(OFFS),)),
        ],
        compiler_params=pltpu.CompilerParams(collective_id=0),
    )(x)


# device time: 23839 ns/iter; 1.7483x vs baseline; 1.0890x over previous
import jax
import jax.numpy as jnp
from jax import lax
from jax.experimental import pallas as pl
from jax.experimental.pallas import tpu as pltpu

N_X, N_Y, N_Z = 2, 2, 4
N_H = 2

OFFS = [
    (dx, dy, dz)
    for dx in range(N_X)
    for dy in range(N_Y)
    for dz in range(N_Z)
    if (dx, dy, dz) != (0, 0, 0)
]


def kernel(x):
    m, n = x.shape
    tm = m // (N_X * N_Y)
    tn = n // N_Z
    hm = tm // N_H

    def body(x_ref, out_ref, rs_buf, red_buf, ag_buf,
             rs_send, rs_recv, ag_send, ag_recv):
        my_x = lax.axis_index("x")
        my_y = lax.axis_index("y")
        my_z = lax.axis_index("z")
        r = 2 * my_x + my_y

        with jax.named_scope("barrier"):
            barrier_sem = pltpu.get_barrier_semaphore()
            for d in range(1, N_Z):
                pl.semaphore_signal(
                    barrier_sem, inc=16,
                    device_id=(my_x, my_y, (my_z + d) % N_Z),
                    device_id_type=pl.DeviceIdType.MESH,
                )
            pl.semaphore_wait(barrier_sem, 16 * (N_Z - 1))
            for dx, dy in ((0, 1), (1, 0), (1, 1)):
                pl.semaphore_signal(
                    barrier_sem, inc=1,
                    device_id=((my_x + dx) % N_X, (my_y + dy) % N_Y, my_z),
                    device_id_type=pl.DeviceIdType.MESH,
                )

        rs = [[None] * (N_Z - 1) for _ in range(N_H)]
        with jax.named_scope("rs_start"):
            for h in range(N_H):
                for d in range(1, N_Z):
                    pz = (my_z + d) % N_Z
                    rdma = pltpu.make_async_remote_copy(
                        src_ref=x_ref.at[
                            pl.ds(r * tm + h * hm, hm), pl.ds(pz * tn, tn)
                        ],
                        dst_ref=rs_buf.at[d - 1, pl.ds(h * hm, hm), :],
                        send_sem=rs_send.at[d - 1, h],
                        recv_sem=rs_recv.at[d - 1, h],
                        device_id=(my_x, my_y, pz),
                        device_id_type=pl.DeviceIdType.MESH,
                    )
                    rdma.start()
                    rs[h][d - 1] = rdma

        ag = [[None] * len(OFFS) for _ in range(N_H)]
        for h in range(N_H):
            hs = pl.ds(h * hm, hm)
            with jax.named_scope(f"rs_wait_h{h}"):
                for rdma in rs[h]:
                    rdma.wait_recv()
            with jax.named_scope(f"reduce_h{h}"):
                red_buf[hs, :] = (
                    x_ref[pl.ds(r * tm + h * hm, hm), pl.ds(my_z * tn, tn)]
                    + rs_buf[0, hs, :] + rs_buf[1, hs, :] + rs_buf[2, hs, :]
                )
                out_ref[pl.ds(r * tm + h * hm, hm), pl.ds(my_z * tn, tn)] = (
                    red_buf[hs, :]
                )
            if h == 0:
                with jax.named_scope("barrier_stage2_wait"):
                    pl.semaphore_wait(barrier_sem, N_X * N_Y - 1)
            with jax.named_scope(f"ag_start_h{h}"):
                for s, (dx, dy, dz) in enumerate(OFFS):
                    tgt = (
                        (my_x + dx) % N_X,
                        (my_y + dy) % N_Y,
                        (my_z + N_Z - dz) % N_Z,
                    )
                    rdma = pltpu.make_async_remote_copy(
                        src_ref=red_buf.at[hs, :],
                        dst_ref=ag_buf.at[s, hs, :],
                        send_sem=ag_send.at[s, h],
                        recv_sem=ag_recv.at[s, h],
                        device_id=tgt,
                        device_id_type=pl.DeviceIdType.MESH,
                    )
                    rdma.start()
                    ag[h][s] = rdma

        for h in range(N_H):
            hs = pl.ds(h * hm, hm)
            with jax.named_scope(f"ag_wait_store_h{h}"):
                for s, (dx, dy, dz) in enumerate(OFFS):
                    ag[h][s].wait_recv()
                    ox = (my_x + dx) % N_X
                    oy = (my_y + dy) % N_Y
                    oz = (my_z + dz) % N_Z
                    orow = 2 * ox + oy
                    out_ref[
                        pl.ds(orow * tm + h * hm, hm), pl.ds(oz * tn, tn)
                    ] = ag_buf[s, hs, :]

        with jax.named_scope("send_cleanup"):
            for h in range(N_H):
                for rdma in rs[h]:
                    rdma.wait_send()
                for rdma in ag[h]:
                    rdma.wait_send()

    return pl.pallas_call(
        body,
        out_shape=jax.ShapeDtypeStruct((m, n), x.dtype),
        in_specs=[pl.BlockSpec(memory_space=pltpu.VMEM)],
        out_specs=pl.BlockSpec(memory_space=pltpu.VMEM),
        scratch_shapes=[
            pltpu.VMEM((N_Z - 1, tm, tn), x.dtype),
            pltpu.VMEM((tm, tn), x.dtype),
            pltpu.VMEM((len(OFFS), tm, tn), x.dtype),
            pltpu.SemaphoreType.DMA((N_Z - 1, N_H)),
            pltpu.SemaphoreType.DMA((N_Z - 1, N_H)),
            pltpu.SemaphoreType.DMA((len(OFFS), N_H)),
            pltpu.SemaphoreType.DMA((len(OFFS), N_H)),
        ],
        compiler_params=pltpu.CompilerParams(collective_id=0),
    )(x)
